# baseline (speedup 1.0000x reference)
import functools

import jax
import jax.numpy as jnp
from jax import lax
from jax.experimental import pallas as pl
from jax.experimental.pallas import tpu as pltpu
from jax.experimental.pallas import tpu_sc as plsc

B = 16
L = 512
E = 128
H = 128
G3 = 3 * H
NTOK = B * L
LABELS = 104


def _gather_rows(table, idx2d):
    info = plsc.get_sparse_core_info()
    nc, ns = info.num_cores, info.num_subcores
    nw = nc * ns
    per_w = NTOK // nw
    ch = 128
    n_ch = per_w // ch
    mesh = plsc.VectorSubcoreMesh(core_axis_name="c", subcore_axis_name="s")

    @functools.partial(
        pl.kernel,
        mesh=mesh,
        out_type=jax.ShapeDtypeStruct((NTOK, E), jnp.float32),
        scratch_types=[
            pltpu.VMEM((n_ch, ch), jnp.int32),
            pltpu.VMEM((per_w, E), jnp.float32),
            pltpu.SemaphoreType.DMA,
        ],
    )
    def gather_k(table_hbm, idx_hbm, out_hbm, idx_v, rows_v, sem):
        wid = lax.axis_index("s") * nc + lax.axis_index("c")
        pltpu.sync_copy(idx_hbm.at[pl.ds(wid * n_ch, n_ch)], idx_v)
        descs = [
            pltpu.async_copy(table_hbm.at[idx_v.at[j]],
                             rows_v.at[pl.ds(j * ch, ch)], sem)
            for j in range(n_ch)
        ]
        for d in descs:
            d.wait()
        pltpu.sync_copy(rows_v, out_hbm.at[pl.ds(wid * per_w, per_w)])

    return gather_k(table, idx2d)


def _tc_body(emb, lens_r, Wc, bc, Wihf, Whhf, bihf, bhhf,
             Wihb, Whhb, bihb, bhhb, Wl, bl, y, gi_scr):
    f32 = jnp.float32
    bf = jnp.bfloat16
    WihfT = Wihf[...].T
    WihbT = Wihb[...].T

    def permc(x):
        return jnp.concatenate(
            [x[:, 0:H], x[:, 2 * H:3 * H], x[:, H:2 * H]], axis=1)

    gsc = jnp.concatenate(
        [jnp.full((1, H), 0.5, f32), jnp.ones((1, H), f32),
         jnp.full((1, H), 0.5, f32)], axis=1)
    gsc2 = jnp.concatenate([gsc, gsc], axis=1)
    A = jnp.concatenate(
        [permc(jnp.dot(Wc[...], WihfT, preferred_element_type=f32)),
         permc(jnp.dot(Wc[...], WihbT, preferred_element_type=f32))],
        axis=1) * gsc2
    A_bf = A.astype(bf)
    def fold_rz(bhh):
        p = permc(bhh) * 0.5
        return jnp.concatenate(
            [p[:, 0:H], jnp.zeros((1, H), f32), p[:, 2 * H:3 * H]], axis=1)

    cvec = (jnp.concatenate(
        [permc(jnp.dot(bc[...], WihfT, preferred_element_type=f32)
               + bihf[...]),
         permc(jnp.dot(bc[...], WihbT, preferred_element_type=f32)
               + bihb[...])],
        axis=1) * gsc2
        + jnp.concatenate([fold_rz(bhhf[...]), fold_rz(bhhb[...])], axis=1))
    chunk = 1024
    for i in range(NTOK // chunk):
        gi_scr[pl.ds(i * chunk, chunk), :] = (
            jnp.dot(emb[pl.ds(i * chunk, chunk), :].astype(bf), A_bf,
                    preferred_element_type=f32) + cvec)

    WhhfT = permc(Whhf[...].T * 0.5).astype(bf)
    WhhbT = permc(Whhb[...].T * 0.5).astype(bf)
    bhnf = permc(bhhf[...])[:, H:2 * H] * 0.5
    bhnb = permc(bhhb[...])[:, H:2 * H] * 0.5
    lens_b = jnp.broadcast_to(lens_r[...], (B, H))
    neg = jnp.float32(-1e9)

    def gru(gi, gh, bhn, h):
        r1 = jnp.tanh(gi[:, 0:H] + gh[:, 0:H])
        zt = jnp.tanh(gi[:, 2 * H:3 * H] + gh[:, 2 * H:3 * H])
        n = jnp.tanh(gi[:, H:2 * H] + (r1 + 1.0) * (gh[:, H:2 * H] + bhn))
        c1 = 0.5 - 0.5 * zt
        c2 = 0.5 + 0.5 * zt
        return c1 * n + c2 * h

    UNROLL = 32
    M = jnp.max(lens_r[...])
    K = (M + (UNROLL - 1)) // UNROLL
    Mp = K * UNROLL

    def substep(t, carry):
        hf, hb, mf, mb = carry
        tb = (Mp - 1) - t
        gif = gi_scr[pl.ds(t * B, B), 0:G3]
        gib = gi_scr[pl.ds(tb * B, B), G3:2 * G3]
        ghf = jnp.dot(hf.astype(bf), WhhfT, preferred_element_type=f32)
        ghb = jnp.dot(hb.astype(bf), WhhbT, preferred_element_type=f32)
        hf2 = gru(gif, ghf, bhnf, hf)
        hb_new = gru(gib, ghb, bhnb, hb)
        vf = t < lens_b
        vb = tb < lens_b
        hb2 = jnp.where(vb, hb_new, hb)
        mf2 = jnp.maximum(mf, jnp.where(vf, hf2, neg))
        mb2 = jnp.maximum(mb, jnp.where(vb, hb2, neg))
        return hf2, hb2, mf2, mb2

    def step(i, carry):
        t0 = i * UNROLL
        for u in range(UNROLL):
            carry = substep(t0 + u, carry)
        return carry

    z0 = jnp.zeros((B, H), f32)
    m0 = jnp.full((B, H), neg, f32)
    _, _, mf, mb = lax.fori_loop(0, K, step, (z0, z0, m0, m0))
    pooled = jnp.concatenate([mf, mb], axis=1)
    y[...] = jnp.dot(pooled, Wl[...], preferred_element_type=f32) + bl[...]


def _tc_call(emb, lens_r, Wc, bc, Wihf, Whhf, bihf, bhhf,
             Wihb, Whhb, bihb, bhhb, Wl, bl, interpret=False):
    return pl.pallas_call(
        _tc_body,
        out_shape=jax.ShapeDtypeStruct((B, LABELS), jnp.float32),
        scratch_shapes=[pltpu.VMEM((NTOK, 2 * G3), jnp.float32)],
        interpret=interpret,
    )(emb, lens_r, Wc, bc, Wihf, Whhf, bihf, bhhf,
      Wihb, Whhb, bihb, bhhb, Wl, bl)


def kernel(tokens, lens, embedding, W_c, b_c, W_ih_f, W_hh_f, b_ih_f,
           b_hh_f, W_ih_b, W_hh_b, b_ih_b, b_hh_b, W_label, b_label):
    idx2d = jnp.transpose(tokens).astype(jnp.int32).reshape(NTOK // 128, 128)
    emb = _gather_rows(embedding, idx2d)
    lens_r = lens.astype(jnp.int32).reshape(B, 1)
    return _tc_call(
        emb, lens_r, W_c, b_c.reshape(1, E),
        W_ih_f, W_hh_f, b_ih_f.reshape(1, G3), b_hh_f.reshape(1, G3),
        W_ih_b, W_hh_b, b_ih_b.reshape(1, G3), b_hh_b.reshape(1, G3),
        W_label, b_label.reshape(1, LABELS))


# speedup vs baseline: 1.0177x; 1.0177x over previous
import functools

import jax
import jax.numpy as jnp
from jax import lax
from jax.experimental import pallas as pl
from jax.experimental.pallas import tpu as pltpu
from jax.experimental.pallas import tpu_sc as plsc

B = 16
L = 512
E = 128
H = 128
G3 = 3 * H
NTOK = B * L
LABELS = 104


def _gather_rows(table, idx2d):
    info = plsc.get_sparse_core_info()
    nc, ns = info.num_cores, info.num_subcores
    nw = nc * ns
    per_w = NTOK // nw
    ch = 128
    n_ch = per_w // ch
    mesh = plsc.VectorSubcoreMesh(core_axis_name="c", subcore_axis_name="s")

    @functools.partial(
        pl.kernel,
        mesh=mesh,
        out_type=jax.ShapeDtypeStruct((NTOK, E), jnp.float32),
        scratch_types=[
            pltpu.VMEM((n_ch, ch), jnp.int32),
            pltpu.VMEM((per_w, E), jnp.float32),
            pltpu.SemaphoreType.DMA,
        ],
    )
    def gather_k(table_hbm, idx_hbm, out_hbm, idx_v, rows_v, sem):
        wid = lax.axis_index("s") * nc + lax.axis_index("c")
        pltpu.sync_copy(idx_hbm.at[pl.ds(wid * n_ch, n_ch)], idx_v)
        descs = [
            pltpu.async_copy(table_hbm.at[idx_v.at[j]],
                             rows_v.at[pl.ds(j * ch, ch)], sem)
            for j in range(n_ch)
        ]
        for d in descs:
            d.wait()
        pltpu.sync_copy(rows_v, out_hbm.at[pl.ds(wid * per_w, per_w)])

    return gather_k(table, idx2d)


def _tc_body(emb, lens_r, Wc, bc, Wihf, Whhf, bihf, bhhf,
             Wihb, Whhb, bihb, bhhb, Wl, bl, y, gi_scr):
    f32 = jnp.float32
    bf = jnp.bfloat16
    WihfT = Wihf[...].T
    WihbT = Wihb[...].T

    def permc(x):
        return jnp.concatenate(
            [x[:, 0:H], x[:, 2 * H:3 * H], x[:, H:2 * H]], axis=1)

    gsc = jnp.concatenate(
        [jnp.full((1, H), 0.5, f32), jnp.ones((1, H), f32),
         jnp.full((1, H), 0.5, f32)], axis=1)
    gsc2 = jnp.concatenate([gsc, gsc], axis=1)
    A = jnp.concatenate(
        [permc(jnp.dot(Wc[...], WihfT, preferred_element_type=f32)),
         permc(jnp.dot(Wc[...], WihbT, preferred_element_type=f32))],
        axis=1) * gsc2
    A_bf = A.astype(bf)
    def fold_rz(bhh):
        p = permc(bhh) * 0.5
        return jnp.concatenate(
            [p[:, 0:H], jnp.zeros((1, H), f32), p[:, 2 * H:3 * H]], axis=1)

    cvec = (jnp.concatenate(
        [permc(jnp.dot(bc[...], WihfT, preferred_element_type=f32)
               + bihf[...]),
         permc(jnp.dot(bc[...], WihbT, preferred_element_type=f32)
               + bihb[...])],
        axis=1) * gsc2
        + jnp.concatenate([fold_rz(bhhf[...]), fold_rz(bhhb[...])], axis=1))
    chunk = 1024
    for i in range(NTOK // chunk):
        gi_scr[pl.ds(i * chunk, chunk), :] = (
            jnp.dot(emb[pl.ds(i * chunk, chunk), :].astype(bf), A_bf,
                    preferred_element_type=f32) + cvec)

    WhhfT = permc(Whhf[...].T * 0.5).astype(bf)
    WhhbT = permc(Whhb[...].T * 0.5).astype(bf)
    bhnf = permc(bhhf[...])[:, H:2 * H] * 0.5
    bhnb = permc(bhhb[...])[:, H:2 * H] * 0.5
    lens_b = jnp.broadcast_to(lens_r[...], (B, H))
    neg = jnp.float32(-1e9)

    def gru(gi, gh, bhn, h):
        r1 = jnp.tanh(gi[:, 0:H] + gh[:, 0:H])
        zt = jnp.tanh(gi[:, 2 * H:3 * H] + gh[:, 2 * H:3 * H])
        n = jnp.tanh(gi[:, H:2 * H] + (r1 + 1.0) * (gh[:, H:2 * H] + bhn))
        c1 = 0.5 - 0.5 * zt
        c2 = 0.5 + 0.5 * zt
        return c1 * n + c2 * h

    UNROLL = 8
    M = jnp.max(lens_r[...])
    K = (M + (UNROLL - 1)) // UNROLL
    Mp = K * UNROLL

    def substep(t, carry):
        hf, hb, mf, mb = carry
        tb = (Mp - 1) - t
        gif = gi_scr[pl.ds(t * B, B), 0:G3]
        gib = gi_scr[pl.ds(tb * B, B), G3:2 * G3]
        ghf = jnp.dot(hf.astype(bf), WhhfT, preferred_element_type=f32)
        ghb = jnp.dot(hb.astype(bf), WhhbT, preferred_element_type=f32)
        hf2 = gru(gif, ghf, bhnf, hf)
        hb_new = gru(gib, ghb, bhnb, hb)
        vf = t < lens_b
        vb = tb < lens_b
        hb2 = jnp.where(vb, hb_new, hb)
        mf2 = jnp.maximum(mf, jnp.where(vf, hf2, neg))
        mb2 = jnp.maximum(mb, jnp.where(vb, hb2, neg))
        return hf2, hb2, mf2, mb2

    def step(i, carry):
        t0 = i * UNROLL
        for u in range(UNROLL):
            carry = substep(t0 + u, carry)
        return carry

    z0 = jnp.zeros((B, H), f32)
    m0 = jnp.full((B, H), neg, f32)
    _, _, mf, mb = lax.fori_loop(0, K, step, (z0, z0, m0, m0))
    pooled = jnp.concatenate([mf, mb], axis=1)
    y[...] = jnp.dot(pooled, Wl[...], preferred_element_type=f32) + bl[...]


def _tc_call(emb, lens_r, Wc, bc, Wihf, Whhf, bihf, bhhf,
             Wihb, Whhb, bihb, bhhb, Wl, bl, interpret=False):
    return pl.pallas_call(
        _tc_body,
        out_shape=jax.ShapeDtypeStruct((B, LABELS), jnp.float32),
        scratch_shapes=[pltpu.VMEM((NTOK, 2 * G3), jnp.float32)],
        interpret=interpret,
    )(emb, lens_r, Wc, bc, Wihf, Whhf, bihf, bhhf,
      Wihb, Whhb, bihb, bhhb, Wl, bl)


def kernel(tokens, lens, embedding, W_c, b_c, W_ih_f, W_hh_f, b_ih_f,
           b_hh_f, W_ih_b, W_hh_b, b_ih_b, b_hh_b, W_label, b_label):
    idx2d = jnp.transpose(tokens).astype(jnp.int32).reshape(NTOK // 128, 128)
    emb = _gather_rows(embedding, idx2d)
    lens_r = lens.astype(jnp.int32).reshape(B, 1)
    return _tc_call(
        emb, lens_r, W_c, b_c.reshape(1, E),
        W_ih_f, W_hh_f, b_ih_f.reshape(1, G3), b_hh_f.reshape(1, G3),
        W_ih_b, W_hh_b, b_ih_b.reshape(1, G3), b_hh_b.reshape(1, G3),
        W_label, b_label.reshape(1, LABELS))
